# Initial kernel scaffold; baseline (speedup 1.0000x reference)
import jax
import jax.numpy as jnp
from jax.experimental import pallas as pl


def kernel(x, edge_index, node_types, hyperedge_types, hyperedge_attr, W_lin, b_lin, W_sheaf, b_sheaf, W1, bW1, bias1, W2, bW2, bias2):
    raise NotImplementedError("write your pallas kernel here")


# speedup vs baseline: 7.0409x; 7.0409x over previous
import functools

import jax
import jax.numpy as jnp
from jax import lax
from jax.experimental import pallas as pl
from jax.experimental.pallas import tpu as pltpu
from jax.experimental.pallas import tpu_sc as plsc

_N = 10000
_NE = 10000
_K = 160000
_F = 128
_HID = 32
_D = 4

_NP = 10240
_RPT = _NP // 16
_KP = 163840
_C = 512
_NW = 32

_mesh = plsc.VectorSubcoreMesh(core_axis_name="c", subcore_axis_name="s")


_BR = 400
_GRID = _N // _BR


def _prep_body(x_ref, hea_ref, wlin_ref, blin_ref, w1bd_ref, bw1t_ref,
               wst_ref, wsb_ref, bsh_ref, h1s_ref, anp_ref, bep_ref):
    h0 = x_ref[...] @ wlin_ref[...] + blin_ref[...]
    he0 = hea_ref[...] @ wlin_ref[...] + blin_ref[...]
    xn = (h0[:, :32] + h0[:, 32:64] + h0[:, 64:96] + h0[:, 96:]) * 0.25
    en = (he0[:, :32] + he0[:, 32:64] + he0[:, 64:96] + he0[:, 96:]) * 0.25
    h1 = h0 @ w1bd_ref[...] + bw1t_ref[...]
    h1s_ref[0] = h1[:, :64]
    h1s_ref[1] = h1[:, 64:]
    anp_ref[...] = xn @ wst_ref[...]
    bep_ref[...] = en @ wsb_ref[...] + bsh_ref[...]


def _prep_call(x, hea, wlin, blin, w1bd, bw1t, wst, wsb, bsh):
    return pl.pallas_call(
        _prep_body,
        grid=(_GRID,),
        in_specs=[
            pl.BlockSpec((_BR, _F), lambda i: (i, 0)),
            pl.BlockSpec((_BR, _F), lambda i: (i, 0)),
            pl.BlockSpec((_F, _F), lambda i: (0, 0)),
            pl.BlockSpec((1, _F), lambda i: (0, 0)),
            pl.BlockSpec((_F, _F), lambda i: (0, 0)),
            pl.BlockSpec((1, _F), lambda i: (0, 0)),
            pl.BlockSpec((_HID, 16), lambda i: (0, 0)),
            pl.BlockSpec((_HID, 16), lambda i: (0, 0)),
            pl.BlockSpec((1, 16), lambda i: (0, 0)),
        ],
        out_specs=[
            pl.BlockSpec((2, _BR, 64), lambda i: (0, i, 0)),
            pl.BlockSpec((_BR, 16), lambda i: (i, 0)),
            pl.BlockSpec((_BR, 16), lambda i: (i, 0)),
        ],
        out_shape=[
            jax.ShapeDtypeStruct((2, _NP, 64), jnp.float32),
            jax.ShapeDtypeStruct((_NP, 16), jnp.float32),
            jax.ShapeDtypeStruct((_NP, 16), jnp.float32),
        ],
    )(x, hea, wlin, blin, w1bd, bw1t, wst, wsb, bsh)


def _mid_body(cs_ref, b1t_ref, w2bd_ref, bw2t_ref, out_ref):
    raw = jnp.concatenate([cs_ref[0], cs_ref[1]], axis=-1) + b1t_ref[...]
    hm = jnp.where(raw > 0, raw, jnp.exp(raw) - 1.0)
    h2 = hm @ w2bd_ref[...] + bw2t_ref[...]
    out_ref[0] = h2[:, :64]
    out_ref[1] = h2[:, 64:]


def _mid_call(cs, b1t, w2bd, bw2t):
    return pl.pallas_call(
        _mid_body,
        grid=(_GRID,),
        in_specs=[
            pl.BlockSpec((2, _BR, 64), lambda i: (0, i, 0)),
            pl.BlockSpec((1, _F), lambda i: (0, 0)),
            pl.BlockSpec((_F, _F), lambda i: (0, 0)),
            pl.BlockSpec((1, _F), lambda i: (0, 0)),
        ],
        out_specs=pl.BlockSpec((2, _BR, 64), lambda i: (0, i, 0)),
        out_shape=jax.ShapeDtypeStruct((2, _NP, 64), jnp.float32),
    )(cs, b1t, w2bd, bw2t)


def _final_body(cs_ref, b2t_ref, out_ref):
    raw = jnp.concatenate([cs_ref[0], cs_ref[1]], axis=-1) + b2t_ref[...]
    out_ref[...] = jnp.where(raw > 0, raw, jnp.exp(raw) - 1.0)


def _final_call(cs, b2t):
    return pl.pallas_call(
        _final_body,
        grid=(_GRID,),
        in_specs=[
            pl.BlockSpec((2, _BR, 64), lambda i: (0, i, 0)),
            pl.BlockSpec((1, _F), lambda i: (0, 0)),
        ],
        out_specs=pl.BlockSpec((_BR, _F), lambda i: (i, 0)),
        out_shape=jax.ShapeDtypeStruct((_N, _F), jnp.float32),
    )(cs, b2t)


@functools.partial(
    pl.kernel,
    out_type=[
        jax.ShapeDtypeStruct((_KP, 16), jnp.float32),
        jax.ShapeDtypeStruct((2, _NP, 16), jnp.float32),
        jax.ShapeDtypeStruct((2, _NP, 16), jnp.float32),
    ],
    mesh=_mesh,
    compiler_params=pltpu.CompilerParams(use_tc_tiling_on_sc=False, needs_layout_passes=False),
    scratch_types=[
        pltpu.VMEM((_C,), jnp.int32),
        pltpu.VMEM((_C,), jnp.int32),
        pltpu.VMEM((_C, 16), jnp.float32),
        pltpu.VMEM((_C, 16), jnp.float32),
        pltpu.VMEM((_C, 16), jnp.float32),
        pltpu.VMEM((_RPT, 16), jnp.float32),
        pltpu.VMEM_SHARED((_NP, 16), jnp.float32),
        pltpu.VMEM_SHARED((_NP, 16), jnp.float32),
        pltpu.SemaphoreType.DMA,
    ],
)
def _sheaf_sc(rowp, colp, anp, bep, alpha_out, dsum_out, bsum_out,
              row_v, col_v, anr_v, ber_v, alpha_v, stage_v,
              dsum_sh, bsum_sh, sem):
    cid = lax.axis_index("c")
    sid = lax.axis_index("s")
    wid = sid * 2 + cid
    kbase = wid * (_KP // _NW)
    rbase = sid * _RPT
    zv = jnp.zeros((16,), jnp.float32)

    def zrow(i, _):
        stage_v[i, :] = zv
        return 0

    lax.fori_loop(0, _RPT, zrow, 0)
    pltpu.sync_copy(stage_v, dsum_sh.at[pl.ds(rbase, _RPT)])
    pltpu.sync_copy(stage_v, bsum_sh.at[pl.ds(rbase, _RPT)])
    plsc.subcore_barrier()

    def chunk(ci, _):
        base = kbase + ci * _C
        pltpu.sync_copy(rowp.at[pl.ds(base, _C)], row_v)
        pltpu.sync_copy(colp.at[pl.ds(base, _C)], col_v)
        pltpu.async_copy(anp.at[row_v], anr_v, sem).wait()
        pltpu.async_copy(bep.at[col_v], ber_v, sem).wait()

        def srow(i, _):
            z = anr_v[i, :] + ber_v[i, :]
            alpha_v[i, :] = 1.0 / (1.0 + jnp.exp(-z))
            return 0

        lax.fori_loop(0, _C, srow, 0)
        pltpu.sync_copy(alpha_v, alpha_out.at[pl.ds(base, _C)])
        pltpu.sync_copy(alpha_v, dsum_sh.at[row_v], add=True)
        pltpu.sync_copy(alpha_v, bsum_sh.at[col_v], add=True)
        return 0

    lax.fori_loop(0, _KP // _NW // _C, chunk, 0)
    plsc.subcore_barrier()

    pltpu.sync_copy(dsum_sh.at[pl.ds(rbase, _RPT)], stage_v)
    pltpu.sync_copy(stage_v, dsum_out.at[cid, pl.ds(rbase, _RPT)])
    pltpu.sync_copy(bsum_sh.at[pl.ds(rbase, _RPT)], stage_v)
    pltpu.sync_copy(stage_v, bsum_out.at[cid, pl.ds(rbase, _RPT)])


_CC = 256
_DR = 128


@functools.partial(
    pl.kernel,
    out_type=jax.ShapeDtypeStruct((2, _NP, 64), jnp.float32),
    mesh=_mesh,
    compiler_params=pltpu.CompilerParams(use_tc_tiling_on_sc=False, needs_layout_passes=False),
    scratch_types=[
        pltpu.VMEM((_CC,), jnp.int32),
        pltpu.VMEM((_CC,), jnp.int32),
        pltpu.VMEM((_CC, 16), jnp.float32),
        pltpu.VMEM((_CC, 64), jnp.float32),
        pltpu.VMEM((_DR, 64), jnp.float32),
        pltpu.VMEM((_DR, 16), jnp.float32),
        pltpu.VMEM((_DR, 16), jnp.float32),
        pltpu.VMEM_SHARED((_NP, 64), jnp.float32),
        pltpu.VMEM_SHARED((_NP, 64), jnp.float32),
        pltpu.SemaphoreType.DMA,
    ],
)
def _conv_sc(rowp, colp, alpha, table, dsum_p, bsum_p, out,
             row_v, col_v, alp_v, g_v, stage_v, inv0_v, inv1_v,
             m_sh, o_sh, sem):
    cid = lax.axis_index("c")
    sid = lax.axis_index("s")
    rbase = sid * _RPT
    kbase = sid * (_KP // 16)
    c0 = cid * 2
    lane = lax.iota(jnp.int32, 16)
    zv = jnp.zeros((16,), jnp.float32)

    def zrow(i, _):
        stage_v[i, pl.ds(0, 16)] = zv
        stage_v[i, pl.ds(16, 16)] = zv
        stage_v[i, pl.ds(32, 16)] = zv
        stage_v[i, pl.ds(48, 16)] = zv
        return 0

    lax.fori_loop(0, _DR, zrow, 0)

    def zinit(q, _):
        roff = rbase + q * _DR
        pltpu.sync_copy(stage_v, m_sh.at[pl.ds(roff, _DR)])
        pltpu.sync_copy(stage_v, o_sh.at[pl.ds(roff, _DR)])
        return 0

    lax.fori_loop(0, _RPT // _DR, zinit, 0)
    plsc.subcore_barrier()

    def scale_rows():
        def grp(g, _):
            iv = g * 16 + lane
            cb0 = plsc.load_gather(alp_v, [iv, jnp.full((16,), c0, jnp.int32)])
            cb1 = plsc.load_gather(alp_v, [iv, jnp.full((16,), c0 + 1, jnp.int32)])
            for f in range(64):
                fv = jnp.full((16,), f, jnp.int32)
                v = plsc.load_gather(g_v, [iv, fv])
                cb = cb0 if f < 32 else cb1
                plsc.store_scatter(g_v, [iv, fv], v * cb)
            return 0

        lax.fori_loop(0, _CC // 16, grp, 0)

    def scale_dense(acc_sh, deg_p, dst=None):
        def dchunk(q, _):
            roff = rbase + q * _DR
            pltpu.sync_copy(acc_sh.at[pl.ds(roff, _DR)], stage_v)
            pltpu.sync_copy(deg_p.at[0, pl.ds(roff, _DR)], inv0_v)
            pltpu.sync_copy(deg_p.at[1, pl.ds(roff, _DR)], inv1_v)

            def inv(i, _):
                v = inv0_v[i, :] + inv1_v[i, :]
                inv0_v[i, :] = jnp.where(v > 0, 1.0 / v, 0.0)
                return 0

            lax.fori_loop(0, _DR, inv, 0)

            def grp(g, _):
                iv = g * 16 + lane
                b0 = plsc.load_gather(inv0_v, [iv, jnp.full((16,), c0, jnp.int32)])
                b1 = plsc.load_gather(inv0_v, [iv, jnp.full((16,), c0 + 1, jnp.int32)])
                for f in range(64):
                    fv = jnp.full((16,), f, jnp.int32)
                    v = plsc.load_gather(stage_v, [iv, fv])
                    b = b0 if f < 32 else b1
                    plsc.store_scatter(stage_v, [iv, fv], v * b)
                return 0

            lax.fori_loop(0, _DR // 16, grp, 0)
            if dst is None:
                pltpu.sync_copy(stage_v, acc_sh.at[pl.ds(roff, _DR)])
            else:
                pltpu.sync_copy(stage_v, dst.at[cid, pl.ds(roff, _DR)])
            return 0

        lax.fori_loop(0, _RPT // _DR, dchunk, 0)

    def mchunk(ci, _):
        base = kbase + ci * _CC
        pltpu.sync_copy(rowp.at[pl.ds(base, _CC)], row_v)
        pltpu.sync_copy(colp.at[pl.ds(base, _CC)], col_v)
        pltpu.sync_copy(alpha.at[pl.ds(base, _CC)], alp_v)
        pltpu.async_copy(table.at[cid].at[row_v], g_v, sem).wait()
        scale_rows()
        pltpu.sync_copy(g_v, m_sh.at[col_v], add=True)
        return 0

    lax.fori_loop(0, _KP // 16 // _CC, mchunk, 0)
    plsc.subcore_barrier()

    scale_dense(m_sh, bsum_p)
    plsc.subcore_barrier()

    def ochunk(ci, _):
        base = kbase + ci * _CC
        pltpu.sync_copy(rowp.at[pl.ds(base, _CC)], row_v)
        pltpu.sync_copy(colp.at[pl.ds(base, _CC)], col_v)
        pltpu.sync_copy(alpha.at[pl.ds(base, _CC)], alp_v)
        pltpu.async_copy(m_sh.at[col_v], g_v, sem).wait()
        scale_rows()
        pltpu.sync_copy(g_v, o_sh.at[row_v], add=True)
        return 0

    lax.fori_loop(0, _KP // 16 // _CC, ochunk, 0)
    plsc.subcore_barrier()

    scale_dense(o_sh, dsum_p, dst=out)


def kernel(x, edge_index, node_types, hyperedge_types, hyperedge_attr,
           W_lin, b_lin, W_sheaf, b_sheaf, W1, bW1, bias1, W2, bW2, bias2):
    del node_types, hyperedge_types
    row = edge_index[0].astype(jnp.int32)
    col = edge_index[1].astype(jnp.int32)
    rowp = jnp.concatenate([row, jnp.full((_KP - _K,), _N, jnp.int32)])
    colp = jnp.concatenate([col, jnp.full((_KP - _K,), _NE, jnp.int32)])

    eye4 = jnp.eye(_D, dtype=jnp.float32)
    w1bd = jnp.kron(eye4, W1)
    w2bd = jnp.kron(eye4, W2)
    bw1t = jnp.tile(bW1, _D)[None, :]
    bw2t = jnp.tile(bW2, _D)[None, :]
    b1t = jnp.tile(bias1, _D)[None, :]
    b2t = jnp.tile(bias2, _D)[None, :]
    wst = jnp.pad(W_sheaf[:_HID], ((0, 0), (0, 12)))
    wsb = jnp.pad(W_sheaf[_HID:], ((0, 0), (0, 12)))
    bsh = jnp.pad(b_sheaf, (0, 12))[None, :]
    blin = b_lin[None, :]

    h1s, anp, bep = _prep_call(x, hyperedge_attr, W_lin, blin, w1bd, bw1t,
                               wst, wsb, bsh)
    alpha, dsum_p, bsum_p = _sheaf_sc(rowp, colp, anp, bep)
    c1 = _conv_sc(rowp, colp, alpha, h1s, dsum_p, bsum_p)
    h2s = _mid_call(c1, b1t, w2bd, bw2t)
    c2 = _conv_sc(rowp, colp, alpha, h2s, dsum_p, bsum_p)
    return _final_call(c2, b2t)


# speedup vs baseline: 25.8644x; 3.6735x over previous
import functools

import jax
import jax.numpy as jnp
from jax import lax
from jax.experimental import pallas as pl
from jax.experimental.pallas import tpu as pltpu
from jax.experimental.pallas import tpu_sc as plsc

_N = 10000
_NE = 10000
_K = 160000
_F = 128
_HID = 32
_D = 4

_NP = 10240
_RPT = _NP // 16
_KP = 163840
_C = 512
_NW = 32

_mesh = plsc.VectorSubcoreMesh(core_axis_name="c", subcore_axis_name="s")


_BR = 400
_GRID = _N // _BR


def _prep_body(x_ref, hea_ref, wlin_ref, blin_ref, w1bd_ref, bw1t_ref,
               wst_ref, wsb_ref, bsh_ref, h1s_ref, anp_ref, bep_ref):
    h0 = x_ref[...] @ wlin_ref[...] + blin_ref[...]
    he0 = hea_ref[...] @ wlin_ref[...] + blin_ref[...]
    xn = (h0[:, :32] + h0[:, 32:64] + h0[:, 64:96] + h0[:, 96:]) * 0.25
    en = (he0[:, :32] + he0[:, 32:64] + he0[:, 64:96] + he0[:, 96:]) * 0.25
    h1 = h0 @ w1bd_ref[...] + bw1t_ref[...]
    h1s_ref[0] = h1[:, :64]
    h1s_ref[1] = h1[:, 64:]
    anp_ref[...] = xn @ wst_ref[...]
    bep_ref[...] = en @ wsb_ref[...] + bsh_ref[...]


def _prep_call(x, hea, wlin, blin, w1bd, bw1t, wst, wsb, bsh):
    return pl.pallas_call(
        _prep_body,
        grid=(_GRID,),
        in_specs=[
            pl.BlockSpec((_BR, _F), lambda i: (i, 0)),
            pl.BlockSpec((_BR, _F), lambda i: (i, 0)),
            pl.BlockSpec((_F, _F), lambda i: (0, 0)),
            pl.BlockSpec((1, _F), lambda i: (0, 0)),
            pl.BlockSpec((_F, _F), lambda i: (0, 0)),
            pl.BlockSpec((1, _F), lambda i: (0, 0)),
            pl.BlockSpec((_HID, 16), lambda i: (0, 0)),
            pl.BlockSpec((_HID, 16), lambda i: (0, 0)),
            pl.BlockSpec((1, 16), lambda i: (0, 0)),
        ],
        out_specs=[
            pl.BlockSpec((2, _BR, 64), lambda i: (0, i, 0)),
            pl.BlockSpec((_BR, 16), lambda i: (i, 0)),
            pl.BlockSpec((_BR, 16), lambda i: (i, 0)),
        ],
        out_shape=[
            jax.ShapeDtypeStruct((2, _NP, 64), jnp.float32),
            jax.ShapeDtypeStruct((_NP, 16), jnp.float32),
            jax.ShapeDtypeStruct((_NP, 16), jnp.float32),
        ],
    )(x, hea, wlin, blin, w1bd, bw1t, wst, wsb, bsh)


def _mid_body(cs_ref, b1t_ref, w2bd_ref, bw2t_ref, out_ref):
    raw = jnp.concatenate([cs_ref[0], cs_ref[1]], axis=-1) + b1t_ref[...]
    hm = jnp.where(raw > 0, raw, jnp.exp(raw) - 1.0)
    h2 = hm @ w2bd_ref[...] + bw2t_ref[...]
    out_ref[0] = h2[:, :64]
    out_ref[1] = h2[:, 64:]


def _mid_call(cs, b1t, w2bd, bw2t):
    return pl.pallas_call(
        _mid_body,
        grid=(_GRID,),
        in_specs=[
            pl.BlockSpec((2, _BR, 64), lambda i: (0, i, 0)),
            pl.BlockSpec((1, _F), lambda i: (0, 0)),
            pl.BlockSpec((_F, _F), lambda i: (0, 0)),
            pl.BlockSpec((1, _F), lambda i: (0, 0)),
        ],
        out_specs=pl.BlockSpec((2, _BR, 64), lambda i: (0, i, 0)),
        out_shape=jax.ShapeDtypeStruct((2, _NP, 64), jnp.float32),
    )(cs, b1t, w2bd, bw2t)


def _final_body(cs_ref, b2t_ref, out_ref):
    raw = jnp.concatenate([cs_ref[0], cs_ref[1]], axis=-1) + b2t_ref[...]
    out_ref[...] = jnp.where(raw > 0, raw, jnp.exp(raw) - 1.0)


def _final_call(cs, b2t):
    return pl.pallas_call(
        _final_body,
        grid=(_GRID,),
        in_specs=[
            pl.BlockSpec((2, _BR, 64), lambda i: (0, i, 0)),
            pl.BlockSpec((1, _F), lambda i: (0, 0)),
        ],
        out_specs=pl.BlockSpec((_BR, _F), lambda i: (i, 0)),
        out_shape=jax.ShapeDtypeStruct((_N, _F), jnp.float32),
    )(cs, b2t)


@functools.partial(
    pl.kernel,
    out_type=[
        jax.ShapeDtypeStruct((_KP, 16), jnp.float32),
        jax.ShapeDtypeStruct((2, _NP, 16), jnp.float32),
        jax.ShapeDtypeStruct((2, _NP, 16), jnp.float32),
    ],
    mesh=_mesh,
    compiler_params=pltpu.CompilerParams(use_tc_tiling_on_sc=False, needs_layout_passes=False),
    scratch_types=[
        pltpu.VMEM((_C,), jnp.int32),
        pltpu.VMEM((_C,), jnp.int32),
        pltpu.VMEM((_C, 16), jnp.float32),
        pltpu.VMEM((_C, 16), jnp.float32),
        pltpu.VMEM((_C, 16), jnp.float32),
        pltpu.VMEM((_RPT, 16), jnp.float32),
        pltpu.VMEM_SHARED((_NP, 16), jnp.float32),
        pltpu.VMEM_SHARED((_NP, 16), jnp.float32),
        pltpu.SemaphoreType.DMA,
    ],
)
def _sheaf_sc(rowp, colp, anp, bep, alpha_out, dsum_out, bsum_out,
              row_v, col_v, anr_v, ber_v, alpha_v, stage_v,
              dsum_sh, bsum_sh, sem):
    cid = lax.axis_index("c")
    sid = lax.axis_index("s")
    wid = sid * 2 + cid
    kbase = wid * (_KP // _NW)
    rbase = sid * _RPT
    zv = jnp.zeros((16,), jnp.float32)

    def zrow(i, _):
        stage_v[i, :] = zv
        return 0

    lax.fori_loop(0, _RPT, zrow, 0)
    pltpu.sync_copy(stage_v, dsum_sh.at[pl.ds(rbase, _RPT)])
    pltpu.sync_copy(stage_v, bsum_sh.at[pl.ds(rbase, _RPT)])
    plsc.subcore_barrier()

    def chunk(ci, _):
        base = kbase + ci * _C
        pltpu.sync_copy(rowp.at[pl.ds(base, _C)], row_v)
        pltpu.sync_copy(colp.at[pl.ds(base, _C)], col_v)
        pltpu.async_copy(anp.at[row_v], anr_v, sem).wait()
        pltpu.async_copy(bep.at[col_v], ber_v, sem).wait()

        def srow(i, _):
            z = anr_v[i, :] + ber_v[i, :]
            alpha_v[i, :] = 1.0 / (1.0 + jnp.exp(-z))
            return 0

        lax.fori_loop(0, _C, srow, 0)
        pltpu.sync_copy(alpha_v, alpha_out.at[pl.ds(base, _C)])
        pltpu.sync_copy(alpha_v, dsum_sh.at[row_v], add=True)
        pltpu.sync_copy(alpha_v, bsum_sh.at[col_v], add=True)
        return 0

    lax.fori_loop(0, _KP // _NW // _C, chunk, 0)
    plsc.subcore_barrier()

    pltpu.sync_copy(dsum_sh.at[pl.ds(rbase, _RPT)], stage_v)
    pltpu.sync_copy(stage_v, dsum_out.at[cid, pl.ds(rbase, _RPT)])
    pltpu.sync_copy(bsum_sh.at[pl.ds(rbase, _RPT)], stage_v)
    pltpu.sync_copy(stage_v, bsum_out.at[cid, pl.ds(rbase, _RPT)])


_CC = 256
_DR = 128


@functools.partial(
    pl.kernel,
    out_type=jax.ShapeDtypeStruct((2, _NP, 64), jnp.float32),
    mesh=_mesh,
    compiler_params=pltpu.CompilerParams(use_tc_tiling_on_sc=False, needs_layout_passes=False),
    scratch_types=[
        pltpu.VMEM((_CC,), jnp.int32),
        pltpu.VMEM((_CC,), jnp.int32),
        pltpu.VMEM((_CC, 16), jnp.float32),
        pltpu.VMEM((_CC, 64), jnp.float32),
        pltpu.VMEM((_CC, 64), jnp.float32),
        pltpu.VMEM((_DR, 16), jnp.float32),
        pltpu.VMEM((_DR, 16), jnp.float32),
        pltpu.VMEM_SHARED((_NP, 64), jnp.float32),
        pltpu.VMEM_SHARED((_NP, 64), jnp.float32),
        pltpu.SemaphoreType.DMA,
    ],
)
def _conv_sc(rowp, colp, alpha, table, dsum_p, bsum_p, out,
             row_v, col_v, alp_v, g_v, g2_v, inv0_v, inv1_v,
             m_sh, o_sh, sem):
    cid = lax.axis_index("c")
    sid = lax.axis_index("s")
    rbase = sid * _RPT
    kbase = sid * (_KP // 16)
    c0 = cid * 2
    lane = lax.iota(jnp.int32, 16)
    zv = jnp.zeros((16,), jnp.float32)

    def zrow(i, _):
        g_v[i, pl.ds(0, 16)] = zv
        g_v[i, pl.ds(16, 16)] = zv
        g_v[i, pl.ds(32, 16)] = zv
        g_v[i, pl.ds(48, 16)] = zv
        return 0

    lax.fori_loop(0, _DR, zrow, 0)

    def zinit(q, _):
        roff = rbase + q * _DR
        pltpu.sync_copy(g_v.at[pl.ds(0, _DR)], m_sh.at[pl.ds(roff, _DR)])
        pltpu.sync_copy(g_v.at[pl.ds(0, _DR)], o_sh.at[pl.ds(roff, _DR)])
        return 0

    lax.fori_loop(0, _RPT // _DR, zinit, 0)
    plsc.subcore_barrier()

    def scale_rows():
        @plsc.parallel_loop(0, _CC, unroll=4)
        def srow(i):
            av = alp_v[i, :]
            cb0 = jnp.where(cid == 0, av[0], av[2])
            cb1 = jnp.where(cid == 0, av[1], av[3])
            g2_v[i, pl.ds(0, 16)] = g_v[i, pl.ds(0, 16)] * cb0
            g2_v[i, pl.ds(16, 16)] = g_v[i, pl.ds(16, 16)] * cb0
            g2_v[i, pl.ds(32, 16)] = g_v[i, pl.ds(32, 16)] * cb1
            g2_v[i, pl.ds(48, 16)] = g_v[i, pl.ds(48, 16)] * cb1

    def scale_dense(acc_sh, deg_p, dst=None):
        def dchunk(q, _):
            roff = rbase + q * _DR
            pltpu.sync_copy(acc_sh.at[pl.ds(roff, _DR)], g_v.at[pl.ds(0, _DR)])
            pltpu.sync_copy(deg_p.at[0, pl.ds(roff, _DR)], inv0_v)
            pltpu.sync_copy(deg_p.at[1, pl.ds(roff, _DR)], inv1_v)

            def inv(i, _):
                v = inv0_v[i, :] + inv1_v[i, :]
                inv0_v[i, :] = jnp.where(v > 0, 1.0 / v, 0.0)
                return 0

            lax.fori_loop(0, _DR, inv, 0)

            @plsc.parallel_loop(0, _DR, unroll=4)
            def drow(i):
                bv = inv0_v[i, :]
                b0 = jnp.where(cid == 0, bv[0], bv[2])
                b1 = jnp.where(cid == 0, bv[1], bv[3])
                g2_v[i, pl.ds(0, 16)] = g_v[i, pl.ds(0, 16)] * b0
                g2_v[i, pl.ds(16, 16)] = g_v[i, pl.ds(16, 16)] * b0
                g2_v[i, pl.ds(32, 16)] = g_v[i, pl.ds(32, 16)] * b1
                g2_v[i, pl.ds(48, 16)] = g_v[i, pl.ds(48, 16)] * b1

            if dst is None:
                pltpu.sync_copy(g2_v.at[pl.ds(0, _DR)], acc_sh.at[pl.ds(roff, _DR)])
            else:
                pltpu.sync_copy(g2_v.at[pl.ds(0, _DR)], dst.at[cid, pl.ds(roff, _DR)])
            return 0

        lax.fori_loop(0, _RPT // _DR, dchunk, 0)

    def mchunk(ci, _):
        base = kbase + ci * _CC
        pltpu.sync_copy(rowp.at[pl.ds(base, _CC)], row_v)
        pltpu.sync_copy(colp.at[pl.ds(base, _CC)], col_v)
        pltpu.sync_copy(alpha.at[pl.ds(base, _CC)], alp_v)
        pltpu.async_copy(table.at[cid].at[row_v], g_v, sem).wait()
        scale_rows()
        pltpu.sync_copy(g2_v, m_sh.at[col_v], add=True)
        return 0

    lax.fori_loop(0, _KP // 16 // _CC, mchunk, 0)
    plsc.subcore_barrier()

    scale_dense(m_sh, bsum_p)
    plsc.subcore_barrier()

    def ochunk(ci, _):
        base = kbase + ci * _CC
        pltpu.sync_copy(rowp.at[pl.ds(base, _CC)], row_v)
        pltpu.sync_copy(colp.at[pl.ds(base, _CC)], col_v)
        pltpu.sync_copy(alpha.at[pl.ds(base, _CC)], alp_v)
        pltpu.async_copy(m_sh.at[col_v], g_v, sem).wait()
        scale_rows()
        pltpu.sync_copy(g2_v, o_sh.at[row_v], add=True)
        return 0

    lax.fori_loop(0, _KP // 16 // _CC, ochunk, 0)
    plsc.subcore_barrier()

    scale_dense(o_sh, dsum_p, dst=out)


def kernel(x, edge_index, node_types, hyperedge_types, hyperedge_attr,
           W_lin, b_lin, W_sheaf, b_sheaf, W1, bW1, bias1, W2, bW2, bias2):
    del node_types, hyperedge_types
    row = edge_index[0].astype(jnp.int32)
    col = edge_index[1].astype(jnp.int32)
    rowp = jnp.concatenate([row, jnp.full((_KP - _K,), _N, jnp.int32)])
    colp = jnp.concatenate([col, jnp.full((_KP - _K,), _NE, jnp.int32)])

    eye4 = jnp.eye(_D, dtype=jnp.float32)
    w1bd = jnp.kron(eye4, W1)
    w2bd = jnp.kron(eye4, W2)
    bw1t = jnp.tile(bW1, _D)[None, :]
    bw2t = jnp.tile(bW2, _D)[None, :]
    b1t = jnp.tile(bias1, _D)[None, :]
    b2t = jnp.tile(bias2, _D)[None, :]
    wst = jnp.pad(W_sheaf[:_HID], ((0, 0), (0, 12)))
    wsb = jnp.pad(W_sheaf[_HID:], ((0, 0), (0, 12)))
    bsh = jnp.pad(b_sheaf, (0, 12))[None, :]
    blin = b_lin[None, :]

    h1s, anp, bep = _prep_call(x, hyperedge_attr, W_lin, blin, w1bd, bw1t,
                               wst, wsb, bsh)
    alpha, dsum_p, bsum_p = _sheaf_sc(rowp, colp, anp, bep)
    c1 = _conv_sc(rowp, colp, alpha, h1s, dsum_p, bsum_p)
    h2s = _mid_call(c1, b1t, w2bd, bw2t)
    c2 = _conv_sc(rowp, colp, alpha, h2s, dsum_p, bsum_p)
    return _final_call(c2, b2t)
